# Initial kernel scaffold; baseline (speedup 1.0000x reference)
import jax
import jax.numpy as jnp
from jax.experimental import pallas as pl


def kernel(pos_ids, k_val, v_val, batch_idx, k_cache, v_cache, page_table):
    raise NotImplementedError("write your pallas kernel here")


# speedup vs baseline: 11.6134x; 11.6134x over previous
import functools

import jax
import jax.numpy as jnp
from jax import lax
from jax.experimental import pallas as pl
from jax.experimental.pallas import tpu as pltpu
from jax.experimental.pallas import tpu_sc as plsc

PAGE = 128


def kernel(pos_ids, k_val, v_val, batch_idx, k_cache, v_cache, page_table):
    B, H, S, D = k_val.shape
    T = k_cache.shape[0]
    NP = T // PAGE
    LP = S // PAGE
    NWT = B * LP
    NPT = NP - NWT

    info = plsc.get_sparse_core_info()
    NC, NS, L = info.num_cores, info.num_subcores, info.num_lanes
    NWK = NC * NS
    WPW = NWT // NWK
    PPW = NPT // NWK
    RPP = PAGE * H
    CH = RPP // L // 8

    lp0 = pos_ids.astype(jnp.int32)[0, ::PAGE] >> 7
    dp = page_table[batch_idx.astype(jnp.int32)[:, None], lp0[None, :]]
    dp_flat = dp.reshape(-1)
    mark = jnp.zeros((NP,), jnp.int32).at[dp_flat].set(1)
    unt = jnp.argsort(mark, stable=True)[:NPT].astype(jnp.int32)

    ar = jnp.arange(RPP, dtype=jnp.int32)
    j_tok, h_head = ar // H, ar % H
    wt = jnp.arange(NWT, dtype=jnp.int32)
    bsrc, slot = wt // LP, wt % LP
    wr_g = (bsrc * (H * S) + slot * PAGE)[:, None] + (h_head * S + j_tok)[None, :]
    wr_o = (dp_flat * RPP)[:, None] + ar[None, :]
    pa = (unt * RPP)[:, None] + ar[None, :]
    wr_g = wr_g.reshape(NWK, WPW, 8, PAGE)
    wr_o = wr_o.reshape(NWK, WPW, 8, PAGE)
    pa = pa.reshape(NWK, PPW, 8, PAGE)

    kvr = k_val.reshape(B * H * S, D)
    vvr = v_val.reshape(B * H * S, D)
    kcr = k_cache.reshape(T * H, D)
    vcr = v_cache.reshape(T * H, D)

    mesh = plsc.VectorSubcoreMesh(core_axis_name="c", subcore_axis_name="s")

    @functools.partial(
        pl.kernel, mesh=mesh,
        out_type=[jax.ShapeDtypeStruct((T * H, D), k_cache.dtype),
                  jax.ShapeDtypeStruct((T * H, D), v_cache.dtype)],
        scratch_types=[
            pltpu.VMEM((8, PAGE), jnp.int32),
            pltpu.VMEM((8, PAGE), jnp.int32),
            pltpu.VMEM((2 * PAGE, D), jnp.float32),
            pltpu.SemaphoreType.DMA,
        ],
    )
    def sc_fill(wrg_hbm, wro_hbm, pa_hbm, kv_hbm, vv_hbm, kc_hbm, vc_hbm,
                ko_hbm, vo_hbm, gix, oix, buf, sem):
        wid = lax.axis_index("s") * NC + lax.axis_index("c")

        def page_task(src_rows, out_rows, g_ref, o_ref):
            for m in range(8):
                half = (m % 2) * PAGE
                pltpu.async_copy(src_rows.at[g_ref.at[m]],
                                 buf.at[pl.ds(half, PAGE)], sem).wait()
                pltpu.async_copy(buf.at[pl.ds(half, PAGE)],
                                 out_rows.at[o_ref.at[m]], sem).wait()

        def written(src_rows, out_rows):
            def body(t, carry):
                pltpu.sync_copy(wrg_hbm.at[wid, t], gix)
                pltpu.sync_copy(wro_hbm.at[wid, t], oix)
                page_task(src_rows, out_rows, gix, oix)
                return carry
            lax.fori_loop(0, WPW, body, 0)

        def passthrough(cache_rows, out_rows):
            def body(t, carry):
                pltpu.sync_copy(pa_hbm.at[wid, t], gix)
                page_task(cache_rows, out_rows, gix, gix)
                return carry
            lax.fori_loop(0, PPW, body, 0)

        written(kv_hbm, ko_hbm)
        passthrough(kc_hbm, ko_hbm)
        written(vv_hbm, vo_hbm)
        passthrough(vc_hbm, vo_hbm)

    ko, vo = sc_fill(wr_g, wr_o, pa, kvr, vvr, kcr, vcr)
    return ko.reshape(T, H, D), vo.reshape(T, H, D)


# speedup vs baseline: 12.8748x; 1.1086x over previous
import functools

import jax
import jax.numpy as jnp
from jax import lax
from jax.experimental import pallas as pl
from jax.experimental.pallas import tpu as pltpu
from jax.experimental.pallas import tpu_sc as plsc

PAGE = 128


def kernel(pos_ids, k_val, v_val, batch_idx, k_cache, v_cache, page_table):
    B, H, S, D = k_val.shape
    T = k_cache.shape[0]
    NP = T // PAGE
    LP = S // PAGE
    NWT = B * LP
    NPT = NP - NWT

    info = plsc.get_sparse_core_info()
    NC, NS, L = info.num_cores, info.num_subcores, info.num_lanes
    NWK = NC * NS
    WPW = NWT // NWK
    PPW = NPT // NWK
    RPP = PAGE * H

    lp0 = pos_ids.astype(jnp.int32)[0, ::PAGE] >> 7
    dp = page_table[batch_idx.astype(jnp.int32)[:, None], lp0[None, :]]
    dp_flat = dp.reshape(-1)
    mark = jnp.zeros((NP,), jnp.int32).at[dp_flat].set(1)
    unt = jnp.argsort(mark, stable=True)[:NPT].astype(jnp.int32)

    ar = jnp.arange(RPP, dtype=jnp.int32)
    j_tok, h_head = ar // H, ar % H
    wt = jnp.arange(NWT, dtype=jnp.int32)
    bsrc, slot = wt // LP, wt % LP
    wr_g = (bsrc * (H * S) + slot * PAGE)[:, None] + (h_head * S + j_tok)[None, :]
    wr_o = (dp_flat * RPP)[:, None] + ar[None, :]
    pa = (unt * RPP)[:, None] + ar[None, :]
    wr_g = wr_g.reshape(NWK, WPW, 8, PAGE)
    wr_o = wr_o.reshape(NWK, WPW, 8, PAGE)
    pa = pa.reshape(NWK, PPW, 8, PAGE)

    kvr = k_val.reshape(B * H * S, D)
    vvr = v_val.reshape(B * H * S, D)
    kcr = k_cache.reshape(T * H, D)
    vcr = v_cache.reshape(T * H, D)

    mesh = plsc.VectorSubcoreMesh(core_axis_name="c", subcore_axis_name="s")

    @functools.partial(
        pl.kernel, mesh=mesh,
        out_type=[jax.ShapeDtypeStruct((T * H, D), k_cache.dtype),
                  jax.ShapeDtypeStruct((T * H, D), v_cache.dtype)],
        scratch_types=[
            pltpu.VMEM((8, PAGE), jnp.int32),
            pltpu.VMEM((8, PAGE), jnp.int32),
            pltpu.VMEM((2 * PAGE, D), jnp.float32),
            pltpu.SemaphoreType.DMA,
            pltpu.SemaphoreType.DMA,
            pltpu.SemaphoreType.DMA,
        ],
    )
    def sc_fill(wrg_hbm, wro_hbm, pa_hbm, kv_hbm, vv_hbm, kc_hbm, vc_hbm,
                ko_hbm, vo_hbm, gix, oix, buf, gsem, s0sem, s1sem):
        wid = lax.axis_index("s") * NC + lax.axis_index("c")
        halves = [buf.at[pl.ds(0, PAGE)], buf.at[pl.ds(PAGE, PAGE)]]
        ssems = [s0sem, s1sem]

        def page_task(src_rows, out_rows, g_ref, o_ref):
            pend = [None, None]
            for m in range(8):
                h = m % 2
                if pend[h] is not None:
                    pend[h].wait()
                pltpu.async_copy(src_rows.at[g_ref.at[m]], halves[h],
                                 gsem).wait()
                pend[h] = pltpu.async_copy(halves[h], out_rows.at[o_ref.at[m]],
                                           ssems[h])
            pend[0].wait()
            pend[1].wait()

        def written(src_rows, out_rows):
            def body(t, carry):
                pltpu.sync_copy(wrg_hbm.at[wid, t], gix)
                pltpu.sync_copy(wro_hbm.at[wid, t], oix)
                page_task(src_rows, out_rows, gix, oix)
                return carry
            lax.fori_loop(0, WPW, body, 0)

        def passthrough(cache_rows, out_rows):
            def body(t, carry):
                pltpu.sync_copy(pa_hbm.at[wid, t], gix)
                page_task(cache_rows, out_rows, gix, gix)
                return carry
            lax.fori_loop(0, PPW, body, 0)

        written(kv_hbm, ko_hbm)
        passthrough(kc_hbm, ko_hbm)
        written(vv_hbm, vo_hbm)
        passthrough(vc_hbm, vo_hbm)

    ko, vo = sc_fill(wr_g, wr_o, pa, kvr, vvr, kcr, vcr)
    return ko.reshape(T, H, D), vo.reshape(T, H, D)


# speedup vs baseline: 16.5982x; 1.2892x over previous
import functools

import jax
import jax.numpy as jnp
from jax import lax
from jax.experimental import pallas as pl
from jax.experimental.pallas import tpu as pltpu
from jax.experimental.pallas import tpu_sc as plsc

PAGE = 128


def kernel(pos_ids, k_val, v_val, batch_idx, k_cache, v_cache, page_table):
    B, H, S, D = k_val.shape
    T = k_cache.shape[0]
    NP = T // PAGE
    LP = S // PAGE
    NWT = B * LP
    NPT = NP - NWT

    info = plsc.get_sparse_core_info()
    NC, NS, L = info.num_cores, info.num_subcores, info.num_lanes
    NWK = NC * NS
    WPW = NWT // NWK
    PPW = NPT // NWK
    RPP = PAGE * H

    lp0 = pos_ids.astype(jnp.int32)[0, ::PAGE] >> 7
    dp = page_table[batch_idx.astype(jnp.int32)[:, None], lp0[None, :]]
    dp_flat = dp.reshape(-1)
    mark = jnp.zeros((NP,), jnp.int32).at[dp_flat].set(1)
    unt = jnp.argsort(mark, stable=True)[:NPT].astype(jnp.int32)

    ar = jnp.arange(RPP, dtype=jnp.int32)
    j_tok, h_head = ar // H, ar % H
    wt = jnp.arange(NWT, dtype=jnp.int32)
    bsrc, slot = wt // LP, wt % LP
    wr_g = (bsrc * (H * S) + slot * PAGE)[:, None] + (h_head * S + j_tok)[None, :]
    wr_o = (dp_flat * RPP)[:, None] + ar[None, :]
    pa = (unt * RPP)[:, None] + ar[None, :]
    wr_g = wr_g.reshape(NWK, WPW, 8, PAGE)
    wr_o = wr_o.reshape(NWK, WPW, 8, PAGE)
    pa = pa.reshape(NWK, PPW, 8, PAGE)

    kvr = k_val.reshape(B * H * S, D)
    vvr = v_val.reshape(B * H * S, D)
    kcr = k_cache.reshape(T * H, D)
    vcr = v_cache.reshape(T * H, D)

    mesh = plsc.VectorSubcoreMesh(core_axis_name="c", subcore_axis_name="s")

    @functools.partial(
        pl.kernel, mesh=mesh,
        out_type=[jax.ShapeDtypeStruct((T * H, D), k_cache.dtype),
                  jax.ShapeDtypeStruct((T * H, D), v_cache.dtype)],
        scratch_types=[
            pltpu.VMEM((8, PAGE), jnp.int32),
            pltpu.VMEM((8, PAGE), jnp.int32),
            pltpu.VMEM((4 * PAGE, D), jnp.float32),
            [pltpu.SemaphoreType.DMA] * 4,
            [pltpu.SemaphoreType.DMA] * 4,
        ],
    )
    def sc_fill(wrg_hbm, wro_hbm, pa_hbm, kv_hbm, vv_hbm, kc_hbm, vc_hbm,
                ko_hbm, vo_hbm, gix, oix, buf, gsems, ssems):
        wid = lax.axis_index("s") * NC + lax.axis_index("c")
        quarters = [buf.at[pl.ds(q * PAGE, PAGE)] for q in range(4)]

        def page_task(src_rows, out_rows, g_ref, o_ref):
            pend_g = [None] * 4
            pend_s = [None] * 4
            for m in range(8):
                q = m % 4
                if pend_s[q] is not None:
                    pend_s[q].wait()
                pend_g[q] = pltpu.async_copy(src_rows.at[g_ref.at[m]],
                                             quarters[q], gsems[q])
                if m >= 1:
                    qp = (m - 1) % 4
                    pend_g[qp].wait()
                    pend_s[qp] = pltpu.async_copy(
                        quarters[qp], out_rows.at[o_ref.at[m - 1]], ssems[qp])
            pend_g[3].wait()
            pend_s[3] = pltpu.async_copy(quarters[3], out_rows.at[o_ref.at[7]],
                                         ssems[3])
            for q in range(4):
                pend_s[q].wait()

        def written(src_rows, out_rows):
            def body(t, carry):
                pltpu.sync_copy(wrg_hbm.at[wid, t], gix)
                pltpu.sync_copy(wro_hbm.at[wid, t], oix)
                page_task(src_rows, out_rows, gix, oix)
                return carry
            lax.fori_loop(0, WPW, body, 0)

        def passthrough(cache_rows, out_rows):
            def body(t, carry):
                pltpu.sync_copy(pa_hbm.at[wid, t], gix)
                page_task(cache_rows, out_rows, gix, gix)
                return carry
            lax.fori_loop(0, PPW, body, 0)

        written(kv_hbm, ko_hbm)
        passthrough(kc_hbm, ko_hbm)
        written(vv_hbm, vo_hbm)
        passthrough(vc_hbm, vo_hbm)

    ko, vo = sc_fill(wr_g, wr_o, pa, kvr, vvr, kcr, vcr)
    return ko.reshape(T, H, D), vo.reshape(T, H, D)


# speedup vs baseline: 17.3054x; 1.0426x over previous
import functools

import jax
import jax.numpy as jnp
from jax import lax
from jax.experimental import pallas as pl
from jax.experimental.pallas import tpu as pltpu
from jax.experimental.pallas import tpu_sc as plsc

PAGE = 128
NRING = 6


def kernel(pos_ids, k_val, v_val, batch_idx, k_cache, v_cache, page_table):
    B, H, S, D = k_val.shape
    T = k_cache.shape[0]
    NP = T // PAGE
    LP = S // PAGE
    NWT = B * LP
    NPT = NP - NWT

    info = plsc.get_sparse_core_info()
    NC, NS, L = info.num_cores, info.num_subcores, info.num_lanes
    NWK = NC * NS
    WPW = NWT // NWK
    PPW = NPT // NWK
    RPP = PAGE * H

    lp0 = pos_ids.astype(jnp.int32)[0, ::PAGE] >> 7
    dp = page_table[batch_idx.astype(jnp.int32)[:, None], lp0[None, :]]
    dp_flat = dp.reshape(-1)
    mark = jnp.zeros((NP,), jnp.int32).at[dp_flat].set(1)
    unt = jnp.argsort(mark, stable=True)[:NPT].astype(jnp.int32)

    ar = jnp.arange(RPP, dtype=jnp.int32)
    j_tok, h_head = ar // H, ar % H
    wt = jnp.arange(NWT, dtype=jnp.int32)
    bsrc, slot = wt // LP, wt % LP
    wr_g = (bsrc * (H * S) + slot * PAGE)[:, None] + (h_head * S + j_tok)[None, :]
    wr_o = (dp_flat * RPP)[:, None] + ar[None, :]
    pa = (unt * RPP)[:, None] + ar[None, :]
    wr_g = wr_g.reshape(NWK, WPW * 8, PAGE)
    wr_o = wr_o.reshape(NWK, WPW * 8, PAGE)
    pa = pa.reshape(NWK, PPW * 8, PAGE)

    kvr = k_val.reshape(B * H * S, D)
    vvr = v_val.reshape(B * H * S, D)
    kcr = k_cache.reshape(T * H, D)
    vcr = v_cache.reshape(T * H, D)

    mesh = plsc.VectorSubcoreMesh(core_axis_name="c", subcore_axis_name="s")

    @functools.partial(
        pl.kernel, mesh=mesh,
        out_type=[jax.ShapeDtypeStruct((T * H, D), k_cache.dtype),
                  jax.ShapeDtypeStruct((T * H, D), v_cache.dtype)],
        scratch_types=[
            pltpu.VMEM((WPW * 8, PAGE), jnp.int32),
            pltpu.VMEM((WPW * 8, PAGE), jnp.int32),
            pltpu.VMEM((PPW * 8, PAGE), jnp.int32),
            pltpu.VMEM((NRING * PAGE, D), jnp.float32),
            [pltpu.SemaphoreType.DMA] * NRING,
            [pltpu.SemaphoreType.DMA] * NRING,
        ],
    )
    def sc_fill(wrg_hbm, wro_hbm, pa_hbm, kv_hbm, vv_hbm, kc_hbm, vc_hbm,
                ko_hbm, vo_hbm, gix, oix, pix, buf, gsems, ssems):
        wid = lax.axis_index("s") * NC + lax.axis_index("c")
        slots = [buf.at[pl.ds(q * PAGE, PAGE)] for q in range(NRING)]
        pltpu.sync_copy(wrg_hbm.at[wid], gix)
        pltpu.sync_copy(wro_hbm.at[wid], oix)
        pltpu.sync_copy(pa_hbm.at[wid], pix)

        chunks = []
        for m in range(WPW * 8):
            chunks.append((kv_hbm, ko_hbm, gix.at[m], oix.at[m]))
        for m in range(PPW * 8):
            chunks.append((kc_hbm, ko_hbm, pix.at[m], pix.at[m]))
        for m in range(WPW * 8):
            chunks.append((vv_hbm, vo_hbm, gix.at[m], oix.at[m]))
        for m in range(PPW * 8):
            chunks.append((vc_hbm, vo_hbm, pix.at[m], pix.at[m]))

        n = len(chunks)
        pend_g = [None] * NRING
        pend_s = [None] * NRING
        for m in range(n):
            q = m % NRING
            if pend_s[q] is not None:
                pend_s[q].wait()
            src, _, gr, _ = chunks[m]
            pend_g[q] = pltpu.async_copy(src.at[gr], slots[q], gsems[q])
            if m >= 1:
                qp = (m - 1) % NRING
                _, out, _, orow = chunks[m - 1]
                pend_g[qp].wait()
                pend_s[qp] = pltpu.async_copy(slots[qp], out.at[orow],
                                              ssems[qp])
        qp = (n - 1) % NRING
        _, out, _, orow = chunks[n - 1]
        pend_g[qp].wait()
        pend_s[qp] = pltpu.async_copy(slots[qp], out.at[orow], ssems[qp])
        for q in range(NRING):
            if pend_s[q] is not None:
                pend_s[q].wait()

    ko, vo = sc_fill(wr_g, wr_o, pa, kvr, vvr, kcr, vcr)
    return ko.reshape(T, H, D), vo.reshape(T, H, D)
